# baseline (speedup 1.0000x reference)
import functools

import jax
import jax.numpy as jnp
from jax import lax
from jax.experimental import pallas as pl
from jax.experimental.pallas import tpu as pltpu
from jax.experimental.pallas import tpu_sc as plsc

B, V, F = 256, 6890, 13776
NC, NS = 2, 16
NW = NC * NS
BPW = B // NW
V3 = V * 3
NPAD_V = 6896
V3P = NPAD_V * 3
VBUF = 20736
NF_IT = F // 16
NZ_IT = V3P // 16
NL_IT = NPAD_V // 16
PAD_CORR = BPW * (NPAD_V - V) / (V * 16.0)


def _rsqrt(x):
    i = plsc.bitcast(x, jnp.int32)
    y = plsc.bitcast(jnp.int32(0x5F3759DF) - (i >> 1), jnp.float32)
    for _ in range(3):
        y = y * (1.5 - 0.5 * x * y * y)
    return y


mesh = plsc.VectorSubcoreMesh(core_axis_name="c", subcore_axis_name="s")


@functools.partial(
    pl.kernel,
    out_type=jax.ShapeDtypeStruct((NW * 16,), jnp.float32),
    mesh=mesh,
    compiler_params=pltpu.CompilerParams(needs_layout_passes=False),
    scratch_types=[
        pltpu.VMEM((F,), jnp.int32),
        pltpu.VMEM((F,), jnp.int32),
        pltpu.VMEM((F,), jnp.int32),
        pltpu.VMEM((VBUF,), jnp.float32),
        pltpu.VMEM((VBUF,), jnp.float32),
        pltpu.VMEM((V3P,), jnp.float32),
        pltpu.VMEM((V3P,), jnp.float32),
        pltpu.VMEM((16,), jnp.float32),
        pltpu.SemaphoreType.DMA,
        pltpu.SemaphoreType.DMA,
    ],
)
def _smpl_normal_loss_sc(pred_hbm, gt_hbm, f0_hbm, f1_hbm, f2_hbm, out_hbm,
                         f0v, f1v, f2v, vp, vg, accp, accg, outv,
                         semp, semg):
    wid = lax.axis_index("s") * NC + lax.axis_index("c")

    def copy_in(t, start, wait):
        b = jnp.minimum(t, BPW - 1) * NW + wid
        cp = pltpu.make_async_copy(pred_hbm.at[b], vp, semp)
        cg = pltpu.make_async_copy(gt_hbm.at[b], vg, semg)
        if start:
            cp.start()
            cg.start()
        if wait:
            cp.wait()
            cg.wait()

    copy_in(0, True, False)
    pltpu.sync_copy(f0_hbm, f0v)
    pltpu.sync_copy(f1_hbm, f1v)
    pltpu.sync_copy(f2_hbm, f2v)

    zeros16 = jnp.zeros((16,), jnp.float32)
    iota3 = lax.iota(jnp.int32, 16) * 3

    @plsc.parallel_loop(0, NZ_IT, unroll=8)
    def zero_body(i):
        accp[pl.ds(i * 16, 16)] = zeros16
        accg[pl.ds(i * 16, 16)] = zeros16

    def batch_body(t, lacc):
        copy_in(t, False, True)

        @plsc.parallel_loop(0, NF_IT, unroll=3)
        def face_body(i):
            j0 = f0v[pl.ds(i * 16, 16)]
            j1 = f1v[pl.ds(i * 16, 16)]
            j2 = f2v[pl.ds(i * 16, 16)]
            j0y, j0z = j0 + 1, j0 + 2
            j1y, j1z = j1 + 1, j1 + 2
            j2y, j2z = j2 + 1, j2 + 2
            for vbuf, acc in ((vp, accp), (vg, accg)):
                v0x = plsc.load_gather(vbuf, [j0])
                v0y = plsc.load_gather(vbuf, [j0y])
                v0z = plsc.load_gather(vbuf, [j0z])
                v1x = plsc.load_gather(vbuf, [j1])
                v1y = plsc.load_gather(vbuf, [j1y])
                v1z = plsc.load_gather(vbuf, [j1z])
                v2x = plsc.load_gather(vbuf, [j2])
                v2y = plsc.load_gather(vbuf, [j2y])
                v2z = plsc.load_gather(vbuf, [j2z])
                ax, ay, az = v1x - v0x, v1y - v0y, v1z - v0z
                bx, by, bz = v2x - v0x, v2y - v0y, v2z - v0z
                nx = ay * bz - az * by
                ny = az * bx - ax * bz
                nz = ax * by - ay * bx
                plsc.addupdate_scatter(acc, [j0], nx)
                plsc.addupdate_scatter(acc, [j1], nx)
                plsc.addupdate_scatter(acc, [j2], nx)
                plsc.addupdate_scatter(acc, [j0y], ny)
                plsc.addupdate_scatter(acc, [j1y], ny)
                plsc.addupdate_scatter(acc, [j2y], ny)
                plsc.addupdate_scatter(acc, [j0z], nz)
                plsc.addupdate_scatter(acc, [j1z], nz)
                plsc.addupdate_scatter(acc, [j2z], nz)

        copy_in(t + 1, True, False)

        @plsc.parallel_loop(0, NL_IT, unroll=2, carry=lacc)
        def loss_body(c, la):
            base = iota3 + c * 48
            base_y, base_z = base + 1, base + 2
            px = plsc.load_gather(accp, [base])
            py = plsc.load_gather(accp, [base_y])
            pz = plsc.load_gather(accp, [base_z])
            gx = plsc.load_gather(accg, [base])
            gy = plsc.load_gather(accg, [base_y])
            gz = plsc.load_gather(accg, [base_z])
            d = px * gx + py * gy + pz * gz
            a2 = px * px + py * py + pz * pz
            b2 = gx * gx + gy * gy + gz * gz
            t2 = jnp.maximum(a2, 1e-16) * jnp.maximum(b2, 1e-16)
            cos = jnp.abs(d) * _rsqrt(t2)
            accp[pl.ds(c * 48, 16)] = zeros16
            accp[pl.ds(c * 48 + 16, 16)] = zeros16
            accp[pl.ds(c * 48 + 32, 16)] = zeros16
            accg[pl.ds(c * 48, 16)] = zeros16
            accg[pl.ds(c * 48 + 16, 16)] = zeros16
            accg[pl.ds(c * 48 + 32, 16)] = zeros16
            return la + (1.0 - cos)

        return loss_body

    lacc = lax.fori_loop(0, BPW, batch_body, zeros16)
    copy_in(BPW - 1, False, True)
    outv[...] = lacc * (1.0 / V) - PAD_CORR
    pltpu.sync_copy(outv, out_hbm.at[pl.ds(wid * 16, 16)])


def kernel(pred_vertices, gt_vertices, faces):
    pad = ((0, 0), (0, VBUF - V3))
    pred = jnp.pad(pred_vertices.reshape(B, V3), pad)
    gt = jnp.pad(gt_vertices.reshape(B, V3), pad)
    f3 = faces * 3
    out = _smpl_normal_loss_sc(pred, gt, f3[:, 0], f3[:, 1], f3[:, 2])
    return jnp.sum(out)


# speedup vs baseline: 1.0038x; 1.0038x over previous
import functools

import jax
import jax.numpy as jnp
from jax import lax
from jax.experimental import pallas as pl
from jax.experimental.pallas import tpu as pltpu
from jax.experimental.pallas import tpu_sc as plsc

B, V, F = 256, 6890, 13776
NC, NS = 2, 16
NW = NC * NS
BPW = B // NW
V3 = V * 3
NPAD_V = 6896
V3P = NPAD_V * 3
VBUF = 20736
NF_IT = F // 16
NZ_IT = V3P // 16
NL_IT = NPAD_V // 16
PAD_CORR = BPW * (NPAD_V - V) / (V * 16.0)


def _rsqrt(x):
    i = plsc.bitcast(x, jnp.int32)
    y = plsc.bitcast(jnp.int32(0x5F3759DF) - (i >> 1), jnp.float32)
    for _ in range(3):
        y = y * (1.5 - 0.5 * x * y * y)
    return y


mesh = plsc.VectorSubcoreMesh(core_axis_name="c", subcore_axis_name="s")


@functools.partial(
    pl.kernel,
    out_type=jax.ShapeDtypeStruct((NW * 16,), jnp.float32),
    mesh=mesh,
    compiler_params=pltpu.CompilerParams(needs_layout_passes=False),
    scratch_types=[
        pltpu.VMEM((F,), jnp.int32),
        pltpu.VMEM((F,), jnp.int32),
        pltpu.VMEM((F,), jnp.int32),
        pltpu.VMEM((VBUF,), jnp.float32),
        pltpu.VMEM((VBUF,), jnp.float32),
        pltpu.VMEM((V3P,), jnp.float32),
        pltpu.VMEM((V3P,), jnp.float32),
        pltpu.VMEM((16,), jnp.float32),
        pltpu.SemaphoreType.DMA,
        pltpu.SemaphoreType.DMA,
    ],
)
def _smpl_normal_loss_sc(pred_hbm, gt_hbm, f0_hbm, f1_hbm, f2_hbm, out_hbm,
                         f0v, f1v, f2v, vp, vg, accp, accg, outv,
                         semp, semg):
    wid = lax.axis_index("s") * NC + lax.axis_index("c")

    def copy_in(t, start, wait):
        b = jnp.minimum(t, BPW - 1) * NW + wid
        cp = pltpu.make_async_copy(pred_hbm.at[b], vp, semp)
        cg = pltpu.make_async_copy(gt_hbm.at[b], vg, semg)
        if start:
            cp.start()
            cg.start()
        if wait:
            cp.wait()
            cg.wait()

    copy_in(0, True, False)
    pltpu.sync_copy(f0_hbm, f0v)
    pltpu.sync_copy(f1_hbm, f1v)
    pltpu.sync_copy(f2_hbm, f2v)

    zeros16 = jnp.zeros((16,), jnp.float32)
    iota3 = lax.iota(jnp.int32, 16) * 3

    @plsc.parallel_loop(0, NZ_IT, unroll=8)
    def zero_body(i):
        accp[pl.ds(i * 16, 16)] = zeros16
        accg[pl.ds(i * 16, 16)] = zeros16

    def batch_body(t, lacc):
        copy_in(t, False, True)

        @plsc.parallel_loop(0, NF_IT, unroll=2)
        def face_body(i):
            j0 = f0v[pl.ds(i * 16, 16)]
            j1 = f1v[pl.ds(i * 16, 16)]
            j2 = f2v[pl.ds(i * 16, 16)]
            j0y, j0z = j0 + 1, j0 + 2
            j1y, j1z = j1 + 1, j1 + 2
            j2y, j2z = j2 + 1, j2 + 2
            for vbuf, acc in ((vp, accp), (vg, accg)):
                v0x = plsc.load_gather(vbuf, [j0])
                v0y = plsc.load_gather(vbuf, [j0y])
                v0z = plsc.load_gather(vbuf, [j0z])
                v1x = plsc.load_gather(vbuf, [j1])
                v1y = plsc.load_gather(vbuf, [j1y])
                v1z = plsc.load_gather(vbuf, [j1z])
                v2x = plsc.load_gather(vbuf, [j2])
                v2y = plsc.load_gather(vbuf, [j2y])
                v2z = plsc.load_gather(vbuf, [j2z])
                ax, ay, az = v1x - v0x, v1y - v0y, v1z - v0z
                bx, by, bz = v2x - v0x, v2y - v0y, v2z - v0z
                nx = ay * bz - az * by
                ny = az * bx - ax * bz
                nz = ax * by - ay * bx
                plsc.addupdate_scatter(acc, [j0], nx)
                plsc.addupdate_scatter(acc, [j1], nx)
                plsc.addupdate_scatter(acc, [j2], nx)
                plsc.addupdate_scatter(acc, [j0y], ny)
                plsc.addupdate_scatter(acc, [j1y], ny)
                plsc.addupdate_scatter(acc, [j2y], ny)
                plsc.addupdate_scatter(acc, [j0z], nz)
                plsc.addupdate_scatter(acc, [j1z], nz)
                plsc.addupdate_scatter(acc, [j2z], nz)

        copy_in(t + 1, True, False)

        @plsc.parallel_loop(0, NL_IT, unroll=2, carry=lacc)
        def loss_body(c, la):
            base = iota3 + c * 48
            base_y, base_z = base + 1, base + 2
            px = plsc.load_gather(accp, [base])
            py = plsc.load_gather(accp, [base_y])
            pz = plsc.load_gather(accp, [base_z])
            gx = plsc.load_gather(accg, [base])
            gy = plsc.load_gather(accg, [base_y])
            gz = plsc.load_gather(accg, [base_z])
            d = px * gx + py * gy + pz * gz
            a2 = px * px + py * py + pz * pz
            b2 = gx * gx + gy * gy + gz * gz
            t2 = jnp.maximum(a2, 1e-16) * jnp.maximum(b2, 1e-16)
            cos = jnp.abs(d) * _rsqrt(t2)
            accp[pl.ds(c * 48, 16)] = zeros16
            accp[pl.ds(c * 48 + 16, 16)] = zeros16
            accp[pl.ds(c * 48 + 32, 16)] = zeros16
            accg[pl.ds(c * 48, 16)] = zeros16
            accg[pl.ds(c * 48 + 16, 16)] = zeros16
            accg[pl.ds(c * 48 + 32, 16)] = zeros16
            return la + (1.0 - cos)

        return loss_body

    lacc = lax.fori_loop(0, BPW, batch_body, zeros16)
    copy_in(BPW - 1, False, True)
    outv[...] = lacc * (1.0 / V) - PAD_CORR
    pltpu.sync_copy(outv, out_hbm.at[pl.ds(wid * 16, 16)])


def kernel(pred_vertices, gt_vertices, faces):
    pad = ((0, 0), (0, VBUF - V3))
    pred = jnp.pad(pred_vertices.reshape(B, V3), pad)
    gt = jnp.pad(gt_vertices.reshape(B, V3), pad)
    f3 = faces * 3
    out = _smpl_normal_loss_sc(pred, gt, f3[:, 0], f3[:, 1], f3[:, 2])
    return jnp.sum(out)
